# Initial kernel scaffold; baseline (speedup 1.0000x reference)
import functools

import jax
import jax.numpy as jnp
from jax import lax
from jax.experimental import pallas as pl
from jax.experimental.pallas import tpu as pltpu
from jax.experimental.pallas import tpu_sc as plsc

N_NODES = 50000
N_EDGES = 1600000

NC = 2
NS = 16
NW = NC * NS

CHUNK = 128
C_STAGE = 8
EPW_STEP = C_STAGE * CHUNK
N_OUT = -(-N_EDGES // (NW * EPW_STEP))
E_PAD = NW * N_OUT * EPW_STEP
N_PAD = 50176
RPS = N_PAD // NS


def _make_sc_agg(F):
  mesh = plsc.VectorSubcoreMesh(core_axis_name="c", subcore_axis_name="s")

  @functools.partial(
      pl.kernel,
      out_type=jax.ShapeDtypeStruct((NC, N_PAD, F), jnp.float32),
      mesh=mesh,
      scratch_types=[
          pltpu.VMEM((C_STAGE, CHUNK), jnp.int32),
          pltpu.VMEM((C_STAGE, CHUNK), jnp.int32),
          pltpu.VMEM((C_STAGE, CHUNK, F), jnp.float32),
          pltpu.VMEM_SHARED((N_PAD, F), jnp.float32),
          pltpu.SemaphoreType.DMA,
      ],
  )
  def agg(ins_hbm, src_hbm, dst_hbm, zeros_hbm, out_hbm,
          sidx, didx, rows, acc, sem):
    c = lax.axis_index("c")
    s = lax.axis_index("s")
    wid = s * NC + c
    pltpu.sync_copy(zeros_hbm, acc.at[pl.ds(s * RPS, RPS)])
    plsc.subcore_barrier()

    @pl.loop(0, N_OUT)
    def _(o):
      blk = (wid * N_OUT + o) * C_STAGE
      pltpu.sync_copy(src_hbm.at[pl.ds(blk, C_STAGE)], sidx)
      pltpu.sync_copy(dst_hbm.at[pl.ds(blk, C_STAGE)], didx)
      cps = []
      for j in range(C_STAGE):
        cps.append(pltpu.async_copy(ins_hbm.at[sidx.at[j]], rows.at[j], sem))
      for j in range(C_STAGE):
        cps[j].wait()
        pltpu.sync_copy(rows.at[j], acc.at[didx.at[j]], add=True)

    plsc.subcore_barrier()
    pltpu.sync_copy(acc.at[pl.ds(s * RPS, RPS)],
                    out_hbm.at[c, pl.ds(s * RPS, RPS)])

  return agg


def _make_sc_degree():
  mesh = plsc.VectorSubcoreMesh(core_axis_name="c", subcore_axis_name="s")

  @functools.partial(
      pl.kernel,
      out_type=jax.ShapeDtypeStruct((NC, N_PAD, 1), jnp.float32),
      mesh=mesh,
      scratch_types=[
          pltpu.VMEM((C_STAGE, CHUNK), jnp.int32),
          pltpu.VMEM((CHUNK, 1), jnp.float32),
          pltpu.VMEM_SHARED((N_PAD, 1), jnp.float32),
      ],
  )
  def deg(dst_hbm, ones_hbm, zeros_hbm, out_hbm, didx, ones_v, acc):
    c = lax.axis_index("c")
    s = lax.axis_index("s")
    wid = s * NC + c
    pltpu.sync_copy(ones_hbm, ones_v)
    pltpu.sync_copy(zeros_hbm, acc.at[pl.ds(s * RPS, RPS)])
    plsc.subcore_barrier()

    @pl.loop(0, N_OUT)
    def _(o):
      blk = (wid * N_OUT + o) * C_STAGE
      pltpu.sync_copy(dst_hbm.at[pl.ds(blk, C_STAGE)], didx)
      for j in range(C_STAGE):
        pltpu.sync_copy(ones_v, acc.at[didx.at[j]], add=True)

    plsc.subcore_barrier()
    pltpu.sync_copy(acc.at[pl.ds(s * RPS, RPS)],
                    out_hbm.at[c, pl.ds(s * RPS, RPS)])

  return deg


_ROWS = 1000
_GRID = N_NODES // _ROWS


def _rowspec(f):
  return pl.BlockSpec((_ROWS, f), lambda i: (i, 0))


def _fullspec(shape):
  return pl.BlockSpec(shape, lambda i: (0, 0))


def _tc_prep(c0_ref, c1_ref, x_ref, s0_ref, dis_ref):
  deg = c0_ref[...] + c1_ref[...] + 1.0
  dis = lax.rsqrt(deg)
  dis_ref[...] = dis
  s0_ref[...] = x_ref[...] * dis


def _tc_layer1(a0_ref, a1_ref, s0_ref, dis_ref, w1_ref, b1_ref, s1_ref):
  pre = dis_ref[...] * (a0_ref[...] + a1_ref[...] + s0_ref[...])
  h = jnp.dot(pre, w1_ref[...], preferred_element_type=jnp.float32)
  h = jnp.maximum(h + b1_ref[...], 0.0)
  s1_ref[...] = h * dis_ref[...]


def _tc_layer2(a0_ref, a1_ref, s1_ref, dis_ref, w2_ref, b2_ref, w3_ref,
               ts_ref):
  pre = dis_ref[...] * (a0_ref[...] + a1_ref[...] + s1_ref[...])
  h = jnp.dot(pre, w2_ref[...], preferred_element_type=jnp.float32)
  h = jnp.maximum(h + b2_ref[...], 0.0)
  t = jnp.dot(h, w3_ref[...], preferred_element_type=jnp.float32)
  ts_ref[...] = t * dis_ref[...]


def _tc_final(a0_ref, a1_ref, ts_ref, dis_ref, b3_ref, out_ref):
  o = dis_ref[...] * (a0_ref[...] + a1_ref[...] + ts_ref[...]) + b3_ref[...]
  m = jnp.max(o, axis=1, keepdims=True)
  lse = m + jnp.log(jnp.sum(jnp.exp(o - m), axis=1, keepdims=True))
  out_ref[...] = o - lse


def kernel(x, edge_index, W1, b1, W2, b2, W3, b3):
  src = edge_index[0].astype(jnp.int32)
  dst = edge_index[1].astype(jnp.int32)
  pad = jnp.full((E_PAD - N_EDGES,), N_NODES, dtype=jnp.int32)
  src2d = jnp.concatenate([src, pad]).reshape(E_PAD // CHUNK, CHUNK)
  dst2d = jnp.concatenate([dst, pad]).reshape(E_PAD // CHUNK, CHUNK)

  zeros1 = jnp.zeros((RPS, 1), jnp.float32)
  ones1 = jnp.ones((CHUNK, 1), jnp.float32)

  counts = _make_sc_degree()(dst2d, ones1, zeros1)
  c0 = counts[0, :N_NODES, :]
  c1 = counts[1, :N_NODES, :]

  s0, dis = pl.pallas_call(
      _tc_prep,
      grid=(_GRID,),
      in_specs=[_rowspec(1), _rowspec(1), _rowspec(16)],
      out_specs=[_rowspec(16), _rowspec(1)],
      out_shape=[
          jax.ShapeDtypeStruct((N_NODES, 16), jnp.float32),
          jax.ShapeDtypeStruct((N_NODES, 1), jnp.float32),
      ],
  )(c0, c1, x)

  pad_row = jnp.zeros((1, 16), jnp.float32)
  agg1 = _make_sc_agg(16)(jnp.concatenate([s0, pad_row]), src2d, dst2d,
                          jnp.zeros((RPS, 16), jnp.float32))

  s1 = pl.pallas_call(
      _tc_layer1,
      grid=(_GRID,),
      in_specs=[_rowspec(16), _rowspec(16), _rowspec(16), _rowspec(1),
                _fullspec((16, 32)), _fullspec((1, 32))],
      out_specs=_rowspec(32),
      out_shape=jax.ShapeDtypeStruct((N_NODES, 32), jnp.float32),
  )(agg1[0, :N_NODES], agg1[1, :N_NODES], s0, dis, W1, b1.reshape(1, 32))

  agg2 = _make_sc_agg(32)(jnp.concatenate([s1, jnp.zeros((1, 32), jnp.float32)]),
                          src2d, dst2d, jnp.zeros((RPS, 32), jnp.float32))

  ts = pl.pallas_call(
      _tc_layer2,
      grid=(_GRID,),
      in_specs=[_rowspec(32), _rowspec(32), _rowspec(32), _rowspec(1),
                _fullspec((32, 64)), _fullspec((1, 64)), _fullspec((64, 2))],
      out_specs=_rowspec(2),
      out_shape=jax.ShapeDtypeStruct((N_NODES, 2), jnp.float32),
  )(agg2[0, :N_NODES], agg2[1, :N_NODES], s1, dis, W2, b2.reshape(1, 64), W3)

  agg3 = _make_sc_agg(2)(jnp.concatenate([ts, jnp.zeros((1, 2), jnp.float32)]),
                         src2d, dst2d, jnp.zeros((RPS, 2), jnp.float32))

  out = pl.pallas_call(
      _tc_final,
      grid=(_GRID,),
      in_specs=[_rowspec(2), _rowspec(2), _rowspec(2), _rowspec(1),
                _fullspec((1, 2))],
      out_specs=_rowspec(2),
      out_shape=jax.ShapeDtypeStruct((N_NODES, 2), jnp.float32),
  )(agg3[0, :N_NODES], agg3[1, :N_NODES], ts, dis, b3.reshape(1, 2))

  return out


# speedup vs baseline: 33.0046x; 33.0046x over previous
import functools

import jax
import jax.numpy as jnp
from jax import lax
from jax.experimental import pallas as pl
from jax.experimental.pallas import tpu as pltpu
from jax.experimental.pallas import tpu_sc as plsc

N_NODES = 50000
N_EDGES = 1600000

NC = 2
NS = 16
NW = NC * NS

CHUNK = 128
C_STAGE = 8
EPW_STEP = C_STAGE * CHUNK
N_OUT = -(-N_EDGES // (NW * EPW_STEP))
E_PAD = NW * N_OUT * EPW_STEP
N_PAD = 50176
RPS = N_PAD // NS


def _make_sc_agg(F):
  mesh = plsc.VectorSubcoreMesh(core_axis_name="c", subcore_axis_name="s")

  @functools.partial(
      pl.kernel,
      out_type=jax.ShapeDtypeStruct((NC, N_PAD, F), jnp.float32),
      mesh=mesh,
      scratch_types=[
          pltpu.VMEM((C_STAGE, CHUNK), jnp.int32),
          pltpu.VMEM((C_STAGE, CHUNK), jnp.int32),
          pltpu.VMEM((C_STAGE, CHUNK, F), jnp.float32),
          pltpu.VMEM_SHARED((N_PAD, F), jnp.float32),
          pltpu.SemaphoreType.DMA,
      ],
      compiler_params=pltpu.CompilerParams(use_tc_tiling_on_sc=False),
  )
  def agg(ins_hbm, src_hbm, dst_hbm, zeros_hbm, out_hbm,
          sidx, didx, rows, acc, sem):
    c = lax.axis_index("c")
    s = lax.axis_index("s")
    wid = s * NC + c
    pltpu.sync_copy(zeros_hbm, acc.at[pl.ds(s * RPS, RPS)])
    plsc.subcore_barrier()

    @pl.loop(0, N_OUT)
    def _(o):
      blk = (wid * N_OUT + o) * C_STAGE
      pltpu.sync_copy(src_hbm.at[pl.ds(blk, C_STAGE)], sidx)
      pltpu.sync_copy(dst_hbm.at[pl.ds(blk, C_STAGE)], didx)
      cps = []
      for j in range(C_STAGE):
        cps.append(pltpu.async_copy(ins_hbm.at[sidx.at[j]], rows.at[j], sem))
      for j in range(C_STAGE):
        cps[j].wait()
      for j in range(C_STAGE):
        pltpu.sync_copy(rows.at[j], acc.at[didx.at[j]], add=True)

    plsc.subcore_barrier()
    pltpu.sync_copy(acc.at[pl.ds(s * RPS, RPS)],
                    out_hbm.at[c, pl.ds(s * RPS, RPS)])

  return agg


def _make_sc_degree():
  mesh = plsc.VectorSubcoreMesh(core_axis_name="c", subcore_axis_name="s")

  @functools.partial(
      pl.kernel,
      out_type=jax.ShapeDtypeStruct((NC, N_PAD, 8), jnp.float32),
      mesh=mesh,
      scratch_types=[
          pltpu.VMEM((C_STAGE, CHUNK), jnp.int32),
          pltpu.VMEM((CHUNK, 8), jnp.float32),
          pltpu.VMEM_SHARED((N_PAD, 8), jnp.float32),
      ],
      compiler_params=pltpu.CompilerParams(use_tc_tiling_on_sc=False),
  )
  def deg(dst_hbm, ones_hbm, zeros_hbm, out_hbm, didx, ones_v, acc):
    c = lax.axis_index("c")
    s = lax.axis_index("s")
    wid = s * NC + c
    pltpu.sync_copy(ones_hbm, ones_v)
    pltpu.sync_copy(zeros_hbm, acc.at[pl.ds(s * RPS, RPS)])
    plsc.subcore_barrier()

    @pl.loop(0, N_OUT)
    def _(o):
      blk = (wid * N_OUT + o) * C_STAGE
      pltpu.sync_copy(dst_hbm.at[pl.ds(blk, C_STAGE)], didx)
      for j in range(C_STAGE):
        pltpu.sync_copy(ones_v, acc.at[didx.at[j]], add=True)

    plsc.subcore_barrier()
    pltpu.sync_copy(acc.at[pl.ds(s * RPS, RPS)],
                    out_hbm.at[c, pl.ds(s * RPS, RPS)])

  return deg


_ROWS = 1000
_GRID = N_NODES // _ROWS


def _rowspec(f):
  return pl.BlockSpec((_ROWS, f), lambda i: (i, 0))


def _fullspec(shape):
  return pl.BlockSpec(shape, lambda i: (0, 0))


def _tc_prep(c0_ref, c1_ref, x_ref, s0_ref, dis_ref):
  deg = c0_ref[...] + c1_ref[...] + 1.0
  dis = lax.rsqrt(deg)
  dis_ref[...] = dis
  s0_ref[...] = x_ref[...] * dis


def _tc_layer1(a0_ref, a1_ref, s0_ref, dis_ref, w1_ref, b1_ref, s1_ref):
  pre = dis_ref[...] * (a0_ref[...] + a1_ref[...] + s0_ref[...])
  h = jnp.dot(pre, w1_ref[...], preferred_element_type=jnp.float32)
  h = jnp.maximum(h + b1_ref[...], 0.0)
  s1_ref[...] = h * dis_ref[...]


def _tc_layer2(h00_ref, h01_ref, h10_ref, h11_ref, s1_ref, dis_ref, w2_ref,
               b2_ref, w3_ref, ts_ref):
  agg = jnp.concatenate(
      [h00_ref[...] + h01_ref[...], h10_ref[...] + h11_ref[...]], axis=1)
  pre = dis_ref[...] * (agg + s1_ref[...])
  h = jnp.dot(pre, w2_ref[...], preferred_element_type=jnp.float32)
  h = jnp.maximum(h + b2_ref[...], 0.0)
  t = jnp.dot(h, w3_ref[...], preferred_element_type=jnp.float32)
  ts_ref[...] = jnp.concatenate(
      [t * dis_ref[...], jnp.zeros_like(t), jnp.zeros_like(t),
       jnp.zeros_like(t)], axis=1)


def _tc_final(a0_ref, a1_ref, ts_ref, dis_ref, b3_ref, out_ref):
  o = dis_ref[...] * (a0_ref[...] + a1_ref[...] + ts_ref[...])[:, :2] \
      + b3_ref[...]
  m = jnp.max(o, axis=1, keepdims=True)
  lse = m + jnp.log(jnp.sum(jnp.exp(o - m), axis=1, keepdims=True))
  out_ref[...] = o - lse


def kernel(x, edge_index, W1, b1, W2, b2, W3, b3):
  src = edge_index[0].astype(jnp.int32)
  dst = edge_index[1].astype(jnp.int32)
  pad = jnp.full((E_PAD - N_EDGES,), N_NODES, dtype=jnp.int32)
  src2d = jnp.concatenate([src, pad]).reshape(E_PAD // CHUNK, CHUNK)
  dst2d = jnp.concatenate([dst, pad]).reshape(E_PAD // CHUNK, CHUNK)

  zeros8 = jnp.zeros((RPS, 8), jnp.float32)
  ones8 = jnp.ones((CHUNK, 8), jnp.float32)

  counts = _make_sc_degree()(dst2d, ones8, zeros8)
  c0 = counts[0, :N_NODES, :1]
  c1 = counts[1, :N_NODES, :1]

  s0, dis = pl.pallas_call(
      _tc_prep,
      grid=(_GRID,),
      in_specs=[_rowspec(1), _rowspec(1), _rowspec(16)],
      out_specs=[_rowspec(16), _rowspec(1)],
      out_shape=[
          jax.ShapeDtypeStruct((N_NODES, 16), jnp.float32),
          jax.ShapeDtypeStruct((N_NODES, 1), jnp.float32),
      ],
  )(c0, c1, x)

  pad_row = jnp.zeros((1, 16), jnp.float32)
  agg1 = _make_sc_agg(16)(jnp.concatenate([s0, pad_row]), src2d, dst2d,
                          jnp.zeros((RPS, 16), jnp.float32))

  s1 = pl.pallas_call(
      _tc_layer1,
      grid=(_GRID,),
      in_specs=[_rowspec(16), _rowspec(16), _rowspec(16), _rowspec(1),
                _fullspec((16, 32)), _fullspec((1, 32))],
      out_specs=_rowspec(32),
      out_shape=jax.ShapeDtypeStruct((N_NODES, 32), jnp.float32),
  )(agg1[0, :N_NODES], agg1[1, :N_NODES], s0, dis, W1, b1.reshape(1, 32))

  s1p0 = jnp.concatenate([s1[:, :16], pad_row])
  s1p1 = jnp.concatenate([s1[:, 16:], pad_row])
  zeros16 = jnp.zeros((RPS, 16), jnp.float32)
  agg2h0 = _make_sc_agg(16)(s1p0, src2d, dst2d, zeros16)
  agg2h1 = _make_sc_agg(16)(s1p1, src2d, dst2d, zeros16)

  ts = pl.pallas_call(
      _tc_layer2,
      grid=(_GRID,),
      in_specs=[_rowspec(16), _rowspec(16), _rowspec(16), _rowspec(16),
                _rowspec(32), _rowspec(1),
                _fullspec((32, 64)), _fullspec((1, 64)), _fullspec((64, 2))],
      out_specs=_rowspec(8),
      out_shape=jax.ShapeDtypeStruct((N_NODES, 8), jnp.float32),
  )(agg2h0[0, :N_NODES], agg2h0[1, :N_NODES],
    agg2h1[0, :N_NODES], agg2h1[1, :N_NODES],
    s1, dis, W2, b2.reshape(1, 64), W3)

  agg3 = _make_sc_agg(8)(jnp.concatenate([ts, jnp.zeros((1, 8), jnp.float32)]),
                         src2d, dst2d, zeros8)

  out = pl.pallas_call(
      _tc_final,
      grid=(_GRID,),
      in_specs=[_rowspec(8), _rowspec(8), _rowspec(8), _rowspec(1),
                _fullspec((1, 2))],
      out_specs=_rowspec(2),
      out_shape=jax.ShapeDtypeStruct((N_NODES, 2), jnp.float32),
  )(agg3[0, :N_NODES], agg3[1, :N_NODES], ts, dis, b3.reshape(1, 2))

  return out
